# baseline (speedup 1.0000x reference)
import functools

import jax
import jax.numpy as jnp
from jax import lax
from jax.experimental import pallas as pl
from jax.experimental.pallas import tpu as pltpu
from jax.experimental.pallas import tpu_sc as plsc

NW = 32
CHUNK = 128
NBUF = 4
DEPTH = NBUF - 1
LANES = 16


def _emb_body(batch, seq, x_hbm, word_hbm, pos_hbm, out_hbm,
              idx_v, word_v, pos_v, gsems, osems):
    c = lax.axis_index("c")
    s = lax.axis_index("s")
    wid = s * 2 + c
    s_per_w = seq // NW
    n_k = s_per_w // CHUNK
    n_chunks = batch * n_k
    s0 = wid * s_per_w

    for b in range(batch):
        pltpu.sync_copy(x_hbm.at[b, pl.ds(s0, s_per_w)], idx_v.at[b])

    def gather(t, buf):
        b, k = t // n_k, t % n_k
        return pltpu.async_copy(
            word_hbm.at[idx_v.at[b, pl.ds(k * CHUNK, CHUNK)]],
            word_v.at[buf], gsems.at[buf])

    pending_in = [None] * NBUF
    pending_out = [None] * NBUF
    for t in range(min(DEPTH, n_chunks)):
        pending_in[t] = gather(t, t)

    pltpu.sync_copy(pos_hbm.at[pl.ds(s0, s_per_w)], pos_v)

    for t in range(n_chunks):
        buf = t % NBUF
        b, k = t // n_k, t % n_k
        nt = t + DEPTH
        if nt < n_chunks:
            nb = nt % NBUF
            if pending_out[nb] is not None:
                pending_out[nb].wait()
                pending_out[nb] = None
            pending_in[nb] = gather(nt, nb)
        pending_in[buf].wait()

        def row_body(i, carry):
            for h in range(8):
                sl = pl.ds(h * LANES, LANES)
                plsc.addupdate(word_v.at[buf, i, sl],
                               pos_v[k * CHUNK + i, sl])
            return carry

        lax.fori_loop(0, CHUNK, row_body, 0)
        pending_out[buf] = pltpu.async_copy(
            word_v.at[buf],
            out_hbm.at[pl.ds(b * seq + s0 + k * CHUNK, CHUNK)], osems.at[buf])

    for buf in range(NBUF):
        if pending_out[buf] is not None:
            pending_out[buf].wait()


@jax.jit
def _run(x, word_table, pos_table):
    batch, seq = x.shape
    vocab, hidden = word_table.shape
    total = batch * seq
    s_per_w = seq // NW

    xi = x.astype(jnp.int32)

    out = pl.kernel(
        functools.partial(_emb_body, batch, seq),
        out_type=jax.ShapeDtypeStruct((total, hidden), jnp.float32),
        mesh=plsc.VectorSubcoreMesh(core_axis_name="c", subcore_axis_name="s"),
        scratch_types=[
            pltpu.VMEM((batch, s_per_w), jnp.int32),
            pltpu.VMEM((NBUF, CHUNK, hidden), jnp.float32),
            pltpu.VMEM((s_per_w, hidden), jnp.float32),
            pltpu.SemaphoreType.DMA((NBUF,)),
            pltpu.SemaphoreType.DMA((NBUF,)),
        ],
    )(xi, word_table, pos_table)
    return out.reshape(batch, seq, hidden)


def kernel(x, word_table, pos_table):
    batch, seq = x.shape
    assert seq % (NW * CHUNK) == 0
    return _run(x, word_table, pos_table)


# speedup vs baseline: 1.0515x; 1.0515x over previous
import functools

import jax
import jax.numpy as jnp
from jax import lax
from jax.experimental import pallas as pl
from jax.experimental.pallas import tpu as pltpu
from jax.experimental.pallas import tpu_sc as plsc

NW = 32
CHUNK = 128
NBUF = 4
DEPTH = NBUF - 1
LANES = 16


def _emb_body(batch, seq, x_hbm, word_hbm, pos_hbm, out_hbm,
              idx_v, word_v, pos_v, gsems, osems, isems, psem):
    c = lax.axis_index("c")
    s = lax.axis_index("s")
    wid = s * 2 + c
    s_per_w = seq // NW
    n_k = s_per_w // CHUNK
    n_chunks = batch * n_k
    s0 = wid * s_per_w

    icopies = [
        pltpu.async_copy(x_hbm.at[b, pl.ds(s0, s_per_w)], idx_v.at[b],
                         isems.at[b])
        for b in range(batch)
    ]
    pos_copy = pltpu.async_copy(pos_hbm.at[pl.ds(s0, s_per_w)], pos_v, psem)
    idx_ready = [False] * batch
    pos_ready = [False]

    def gather(t, buf):
        b, k = t // n_k, t % n_k
        if not idx_ready[b]:
            icopies[b].wait()
            idx_ready[b] = True
        return pltpu.async_copy(
            word_hbm.at[idx_v.at[b, pl.ds(k * CHUNK, CHUNK)]],
            word_v.at[buf], gsems.at[buf])

    pending_in = [None] * NBUF
    pending_out = [None] * NBUF
    for t in range(min(DEPTH, n_chunks)):
        pending_in[t] = gather(t, t)

    for t in range(n_chunks):
        buf = t % NBUF
        b, k = t // n_k, t % n_k
        nt = t + DEPTH
        if nt < n_chunks:
            nb = nt % NBUF
            if pending_out[nb] is not None:
                pending_out[nb].wait()
                pending_out[nb] = None
            pending_in[nb] = gather(nt, nb)
        pending_in[buf].wait()
        if not pos_ready[0]:
            pos_copy.wait()
            pos_ready[0] = True

        def row_body(i, carry):
            for h in range(8):
                sl = pl.ds(h * LANES, LANES)
                plsc.addupdate(word_v.at[buf, i, sl],
                               pos_v[k * CHUNK + i, sl])
            return carry

        lax.fori_loop(0, CHUNK, row_body, 0)
        pending_out[buf] = pltpu.async_copy(
            word_v.at[buf],
            out_hbm.at[pl.ds(b * seq + s0 + k * CHUNK, CHUNK)], osems.at[buf])

    for buf in range(NBUF):
        if pending_out[buf] is not None:
            pending_out[buf].wait()


@jax.jit
def _run(x, word_table, pos_table):
    batch, seq = x.shape
    vocab, hidden = word_table.shape
    total = batch * seq
    s_per_w = seq // NW

    xi = x.astype(jnp.int32)

    out = pl.kernel(
        functools.partial(_emb_body, batch, seq),
        out_type=jax.ShapeDtypeStruct((total, hidden), jnp.float32),
        mesh=plsc.VectorSubcoreMesh(core_axis_name="c", subcore_axis_name="s"),
        scratch_types=[
            pltpu.VMEM((batch, s_per_w), jnp.int32),
            pltpu.VMEM((NBUF, CHUNK, hidden), jnp.float32),
            pltpu.VMEM((s_per_w, hidden), jnp.float32),
            pltpu.SemaphoreType.DMA((NBUF,)),
            pltpu.SemaphoreType.DMA((NBUF,)),
            pltpu.SemaphoreType.DMA((batch,)),
            pltpu.SemaphoreType.DMA,
        ],
    )(xi, word_table, pos_table)
    return out.reshape(batch, seq, hidden)


def kernel(x, word_table, pos_table):
    batch, seq = x.shape
    assert seq % (NW * CHUNK) == 0
    return _run(x, word_table, pos_table)
